# baseline (speedup 1.0000x reference)
import functools

import jax
import jax.numpy as jnp
from jax import lax
from jax.experimental import pallas as pl
from jax.experimental.pallas import tpu as pltpu
from jax.experimental.pallas import tpu_sc as plsc

NC = 2
NS = 16
NW = NC * NS
CW = 128


@functools.partial(jax.jit, static_argnames=())
def _sc_embedding_gather(idx3, table):
    nw, nchunk, cw = idx3.shape
    v, d = table.shape
    b_total = nw * nchunk * cw

    mesh = plsc.VectorSubcoreMesh(core_axis_name="c", subcore_axis_name="s")

    @functools.partial(
        pl.kernel,
        mesh=mesh,
        compiler_params=pltpu.CompilerParams(use_tc_tiling_on_sc=False),
        out_type=jax.ShapeDtypeStruct((b_total, d), jnp.float32),
        scratch_types=[
            pltpu.VMEM((nchunk, cw), jnp.int32),
            pltpu.VMEM((cw, d), jnp.float32),
            pltpu.SemaphoreType.DMA,
        ],
    )
    def k(idx_hbm, table_hbm, out_hbm, idx_v, rows_v, gsem):
        wid = lax.axis_index("s") * NC + lax.axis_index("c")
        pltpu.sync_copy(idx_hbm.at[wid], idx_v)
        base = wid * nchunk

        def chunk_body(j, carry):
            pltpu.async_copy(table_hbm.at[idx_v.at[j]], rows_v, gsem).wait()
            pltpu.sync_copy(rows_v, out_hbm.at[pl.ds((base + j) * cw, cw)])
            return carry

        lax.fori_loop(0, nchunk, chunk_body, 0)

    return k(idx3, table)


def kernel(x, embedding):
    bsz, hist = x.shape
    d = embedding.shape[1]
    b_total = bsz * hist
    nchunk = b_total // (NW * CW)
    idx3 = x.astype(jnp.int32).reshape(NW, nchunk, CW)
    out = _sc_embedding_gather(idx3, embedding)
    return out.reshape(bsz, hist, d)


# speedup vs baseline: 1.1133x; 1.1133x over previous
import functools

import jax
import jax.numpy as jnp
from jax import lax
from jax.experimental import pallas as pl
from jax.experimental.pallas import tpu as pltpu
from jax.experimental.pallas import tpu_sc as plsc

NC = 2
NS = 16
NW = NC * NS
CW = 128
NBUF = 8
LEAD = 4


def _sc_embedding_gather(idx3, table):
    nw, nchunk, cw = idx3.shape
    v, d = table.shape
    b_total = nw * nchunk * cw
    nsteps = nchunk // NBUF
    assert nchunk % NBUF == 0 and nsteps >= 2 and LEAD < NBUF

    mesh = plsc.VectorSubcoreMesh(core_axis_name="c", subcore_axis_name="s")

    @functools.partial(
        pl.kernel,
        mesh=mesh,
        compiler_params=pltpu.CompilerParams(use_tc_tiling_on_sc=False),
        out_type=jax.ShapeDtypeStruct((b_total, d), jnp.float32),
        scratch_types=(
            [pltpu.VMEM((nchunk, cw), jnp.int32),
             pltpu.VMEM((NBUF, cw, d), jnp.float32)]
            + [pltpu.SemaphoreType.DMA] * (2 * NBUF)
        ),
    )
    def k(idx_hbm, table_hbm, out_hbm, idx_v, rows_v, *sems):
        gsem = sems[:NBUF]
        osem = sems[NBUF:]
        wid = lax.axis_index("s") * NC + lax.axis_index("c")
        pltpu.sync_copy(idx_hbm.at[wid], idx_v)
        base = wid * nchunk

        def start_gather(j, b):
            pltpu.async_copy(table_hbm.at[idx_v.at[j]], rows_v.at[b], gsem[b])

        def wait_gather(j, b):
            pltpu.make_async_copy(
                table_hbm.at[idx_v.at[j]], rows_v.at[b], gsem[b]).wait()

        def out_slice(j):
            return out_hbm.at[pl.ds((base + j) * cw, cw)]

        def start_ocopy(j, b):
            pltpu.async_copy(rows_v.at[b], out_slice(j), osem[b])

        def wait_ocopy(j, b):
            pltpu.make_async_copy(rows_v.at[b], out_slice(j), osem[b]).wait()

        for bb in range(LEAD):
            start_gather(bb, bb)

        for bb in range(NBUF):
            bg = (bb + LEAD) % NBUF
            if bb + LEAD >= NBUF:
                wait_ocopy(bb + LEAD - NBUF, bg)
            start_gather(bb + LEAD, bg)
            wait_gather(bb, bb)
            start_ocopy(bb, bb)

        def round_body(s, carry):
            for bb in range(NBUF):
                j = s * NBUF + bb
                bg = (bb + LEAD) % NBUF
                wait_ocopy(j + LEAD - NBUF, bg)
                start_gather(j + LEAD, bg)
                wait_gather(j, bb)
                start_ocopy(j, bb)
            return carry

        lax.fori_loop(1, nsteps - 1, round_body, 0)

        for bb in range(NBUF):
            j = (nsteps - 1) * NBUF + bb
            bg = (bb + LEAD) % NBUF
            if bb + LEAD < NBUF:
                wait_ocopy(j + LEAD - NBUF, bg)
                start_gather(j + LEAD, bg)
            wait_gather(j, bb)
            start_ocopy(j, bb)

        for bb in range(NBUF):
            wait_ocopy((nsteps - 1) * NBUF + bb, bb)

    return k(idx3, table)


def kernel(x, embedding):
    bsz, hist = x.shape
    d = embedding.shape[1]
    b_total = bsz * hist
    nchunk = b_total // (NW * CW)
    idx3 = x.astype(jnp.int32).reshape(NW, nchunk, CW)
    out = _sc_embedding_gather(idx3, embedding)
    return out.reshape(bsz, hist, d)
